# Initial kernel scaffold; baseline (speedup 1.0000x reference)
import jax
import jax.numpy as jnp
from jax.experimental import pallas as pl


def kernel(edge_index, h_user, h_item):
    raise NotImplementedError("write your pallas kernel here")


# speedup vs baseline: 3.5018x; 3.5018x over previous
import functools

import jax
import jax.numpy as jnp
from jax import lax
from jax.experimental import pallas as pl
from jax.experimental.pallas import tpu as pltpu
from jax.experimental.pallas import tpu_sc as plsc

N_EDGES = 320000
N_WORKERS = 32
CHUNK = 80
D = 128
L = 16


def _sc_body(src_hbm, dst_hbm, hu_hbm, hi_hbm, out_hbm,
             src_v, dst_v, u_v, v_v, part_v, out_v, sem_u, sem_v):
    per_w = N_EDGES // N_WORKERS
    n_chunks = per_w // CHUNK
    wid = lax.axis_index("s") * 2 + lax.axis_index("c")
    base = wid * per_w

    def chunk_body(i, carry):
        off = base + i * CHUNK
        pltpu.sync_copy(src_hbm.at[pl.ds(off, CHUNK)], src_v)
        pltpu.sync_copy(dst_hbm.at[pl.ds(off, CHUNK)], dst_v)
        cu = pltpu.async_copy(hu_hbm.at[src_v], u_v, sem_u)
        cv = pltpu.async_copy(hi_hbm.at[dst_v], v_v, sem_v)
        cu.wait()
        cv.wait()

        def edge_body(c, carry2):
            acc = u_v[c, pl.ds(0, L)] * v_v[c, pl.ds(0, L)]
            for k in range(1, D // L):
                acc = acc + u_v[c, pl.ds(k * L, L)] * v_v[c, pl.ds(k * L, L)]
            part_v[pl.ds(c * L, L)] = acc
            return carry2

        lax.fori_loop(0, CHUNK, edge_body, 0)

        def group_body(g, carry2):
            rowbase = (g * L + lax.iota(jnp.int32, L)) * L
            s = plsc.load_gather(part_v, [rowbase])
            for j in range(1, L):
                s = s + plsc.load_gather(part_v, [rowbase + j])
            out_v[pl.ds(g * L, L)] = s
            return carry2

        lax.fori_loop(0, CHUNK // L, group_body, 0)
        pltpu.sync_copy(out_v, out_hbm.at[pl.ds(off, CHUNK)])
        return carry

    lax.fori_loop(0, n_chunks, chunk_body, 0)


@jax.jit
def _run(src, dst, h_user, h_item):
    mesh = plsc.VectorSubcoreMesh(core_axis_name="c", subcore_axis_name="s")
    f = functools.partial(
        pl.kernel,
        out_type=jax.ShapeDtypeStruct((N_EDGES,), jnp.float32),
        mesh=mesh,
        scratch_types=[
            pltpu.VMEM((CHUNK,), jnp.int32),
            pltpu.VMEM((CHUNK,), jnp.int32),
            pltpu.VMEM((CHUNK, D), jnp.float32),
            pltpu.VMEM((CHUNK, D), jnp.float32),
            pltpu.VMEM((CHUNK * L,), jnp.float32),
            pltpu.VMEM((CHUNK,), jnp.float32),
            pltpu.SemaphoreType.DMA,
            pltpu.SemaphoreType.DMA,
        ],
        compiler_params=pltpu.CompilerParams(needs_layout_passes=False),
    )(_sc_body)
    return f(src, dst, h_user, h_item)


def kernel(edge_index, h_user, h_item):
    ei = edge_index.astype(jnp.int32)
    return _run(ei[0], ei[1], h_user, h_item)
